# Initial kernel scaffold; baseline (speedup 1.0000x reference)
import jax
import jax.numpy as jnp
from jax.experimental import pallas as pl


def kernel(x):
    raise NotImplementedError("write your pallas kernel here")


# speedup vs baseline: 61.7227x; 61.7227x over previous
import functools

import jax
import jax.numpy as jnp
from jax import lax
from jax.experimental import pallas as pl
from jax.experimental.pallas import tpu as pltpu
from jax.experimental.pallas import tpu_sc as plsc

_NC, _NS, _L = 2, 16, 16
_NW = _NC * _NS
_C = 1048576
_NROWS = 12

_BC = 16384
_NT = 13
_CT = _NT * _BC
_CS = _C - _CT
_CW = _CS // _NW
assert _CW % 8 == 0 and _CW % _L == 0
_UNROLL = 8
_NVEC = _CW // _L

_mesh = plsc.VectorSubcoreMesh(
    core_axis_name="c", subcore_axis_name="s",
    num_cores=_NC, num_subcores=_NS)

_ABS_MASK = 0x7FFFFFFF
_INF_BITS = 0x7F800000


def _row_addr(k):
    q = k // 4
    m = k % 4
    b = m + 1 + jnp.where((m >= 2) & (q > 0), 1, 0)
    return q, b


@functools.partial(
    pl.kernel,
    out_type=jax.ShapeDtypeStruct((_NW, _L), jnp.int32),
    mesh=_mesh,
    scratch_types=[
        pltpu.VMEM((_CW,), jnp.float32),
        pltpu.VMEM((_CW,), jnp.float32),
        pltpu.VMEM((_L,), jnp.int32),
        pltpu.SemaphoreType.DMA,
        pltpu.SemaphoreType.DMA,
    ],
)
def _sc_masked_selfcmp(x_hbm, out_hbm, buf0, buf1, accv, sem0, sem1):
    cid = lax.axis_index("c")
    sid = lax.axis_index("s")
    wid = sid * _NC + cid
    colbase = _CT + wid * _CW

    absmask = jnp.full((_L,), _ABS_MASK, jnp.int32)

    def start(k, buf, sem):
        a, b = _row_addr(k)
        pltpu.async_copy(x_hbm.at[a, b, pl.ds(colbase, _CW)], buf, sem)

    def drain(buf, sem):
        pltpu.make_async_copy(
            x_hbm.at[0, 1, pl.ds(colbase, _CW)], buf, sem).wait()

    def scan(buf, acc):
        def body(j, acc):
            base = j * (_L * _UNROLL)
            for u in range(_UNROLL):
                v = buf[pl.ds(base + u * _L, _L)]
                bits = lax.bitcast_convert_type(v, jnp.int32) & absmask
                acc = jnp.maximum(acc, bits)
            return acc
        return lax.fori_loop(0, _NVEC // _UNROLL, body, acc)

    start(jnp.int32(0), buf0, sem0)

    def pair(g, acc):
        start(2 * g + 1, buf1, sem1)
        drain(buf0, sem0)
        acc = scan(buf0, acc)

        @pl.when(g < _NROWS // 2 - 1)
        def _():
            start(2 * g + 2, buf0, sem0)

        drain(buf1, sem1)
        return scan(buf1, acc)

    acc = lax.fori_loop(0, _NROWS // 2, pair, jnp.zeros((_L,), jnp.int32))

    accv[...] = acc
    pltpu.sync_copy(accv, out_hbm.at[wid])


def _tc_body(x_ref, o_ref, acc_ref):
    i = pl.program_id(0)
    j = pl.program_id(1)

    @pl.when((i == 0) & (j == 0))
    def _():
        acc_ref[...] = jnp.zeros((8, 6, 128), jnp.int32)

    rows = lax.broadcasted_iota(jnp.int32, (6, 1), 0)
    is0 = (i == 0)
    rowmask = ((rows != 0)
               & ((rows != 5) | jnp.logical_not(is0))
               & ((rows != 3) | is0))
    accs = [acc_ref[k] for k in range(8)]
    for t in range(_BC // 128):
        chunk = x_ref[0, :, t * 128:(t + 1) * 128]
        bits = lax.bitcast_convert_type(chunk, jnp.int32) & jnp.int32(_ABS_MASK)
        k = t % 8
        accs[k] = jnp.maximum(accs[k], jnp.where(rowmask, bits, 0))
    for k in range(8):
        acc_ref[k] = accs[k]

    @pl.when((i == 2) & (j == _NT - 1))
    def _():
        o_ref[0, 0] = jnp.max(acc_ref[...])


def _tc_masked_selfcmp(x):
    return pl.pallas_call(
        _tc_body,
        grid=(3, _NT),
        in_specs=[pl.BlockSpec((1, 6, _BC), lambda i, j: (i, 0, j))],
        out_specs=pl.BlockSpec(
            (1, 1), lambda i, j: (0, 0), memory_space=pltpu.SMEM),
        out_shape=jax.ShapeDtypeStruct((1, 1), jnp.int32),
        scratch_shapes=[pltpu.VMEM((8, 6, 128), jnp.int32)],
    )(x)


def kernel(x):
    sc_parts = _sc_masked_selfcmp(x)
    tc_part = _tc_masked_selfcmp(x)
    worst = jnp.maximum(jnp.max(sc_parts), tc_part[0, 0])
    return worst <= jnp.int32(_INF_BITS)


# speedup vs baseline: 71.6306x; 1.1605x over previous
import functools

import jax
import jax.numpy as jnp
from jax import lax
from jax.experimental import pallas as pl
from jax.experimental.pallas import tpu as pltpu
from jax.experimental.pallas import tpu_sc as plsc

_NC, _NS, _L = 2, 16, 16
_NW = _NC * _NS
_C = 1048576
_NROWS = 12

_CW = _C // _NW
_CH = _CW // 2
_NQ = 2 * _NROWS
_NB = 4
assert _CH % 8 == 0 and _CH % _L == 0
_UNROLL = 8

_mesh = plsc.VectorSubcoreMesh(
    core_axis_name="c", subcore_axis_name="s",
    num_cores=_NC, num_subcores=_NS)

_ABS_MASK = 0x7FFFFFFF
_INF_BITS = 0x7F800000


def _row_addr(k):
    q = k // 4
    m = k % 4
    b = m + 1 + jnp.where((m >= 2) & (q > 0), 1, 0)
    return q, b


@functools.partial(
    pl.kernel,
    out_type=jax.ShapeDtypeStruct((_NW, _L), jnp.int32),
    mesh=_mesh,
    scratch_types=[
        pltpu.VMEM((_CH,), jnp.float32),
        pltpu.VMEM((_CH,), jnp.float32),
        pltpu.VMEM((_CH,), jnp.float32),
        pltpu.VMEM((_CH,), jnp.float32),
        pltpu.VMEM((_L,), jnp.int32),
        pltpu.SemaphoreType.DMA,
        pltpu.SemaphoreType.DMA,
        pltpu.SemaphoreType.DMA,
        pltpu.SemaphoreType.DMA,
    ],
)
def _sc_masked_selfcmp(x_hbm, out_hbm, b0, b1, b2, b3, accv,
                       s0, s1, s2, s3):
    cid = lax.axis_index("c")
    sid = lax.axis_index("s")
    wid = sid * _NC + cid
    colbase = wid * _CW
    bufs = (b0, b1, b2, b3)
    sems = (s0, s1, s2, s3)

    absmask = jnp.full((_L,), _ABS_MASK, jnp.int32)

    def start(q, slot):
        a, b = _row_addr(q >> 1)
        cb = colbase + (q & 1) * _CH
        pltpu.async_copy(
            x_hbm.at[a, b, pl.ds(cb, _CH)], bufs[slot], sems[slot])

    def drain(slot):
        pltpu.make_async_copy(
            x_hbm.at[0, 1, pl.ds(colbase, _CH)],
            bufs[slot], sems[slot]).wait()

    def scan(slot, acc):
        buf = bufs[slot]

        def body(j, acc):
            base = j * (_L * _UNROLL)
            for u in range(_UNROLL):
                v = buf[pl.ds(base + u * _L, _L)]
                bits = lax.bitcast_convert_type(v, jnp.int32) & absmask
                acc = jnp.maximum(acc, bits)
            return acc
        return lax.fori_loop(0, _CH // (_L * _UNROLL), body, acc)

    for q in range(_NB - 1):
        start(jnp.int32(q), q)

    def step(g, acc):
        q0 = _NB * g
        for u in range(_NB):
            drain(u)

            @pl.when(q0 + u + (_NB - 1) < _NQ)
            def _(q0=q0, u=u):
                start(q0 + u + (_NB - 1), (u + _NB - 1) % _NB)

            acc = scan(u, acc)
        return acc

    acc = lax.fori_loop(0, _NQ // _NB, step, jnp.zeros((_L,), jnp.int32))

    accv[...] = acc
    pltpu.sync_copy(accv, out_hbm.at[wid])


def kernel(x):
    partials = _sc_masked_selfcmp(x)
    return jnp.max(partials) <= jnp.int32(_INF_BITS)
